# baseline (speedup 1.0000x reference)
import jax
import jax.numpy as jnp
from jax.experimental import pallas as pl
from jax.experimental.pallas import tpu as pltpu

_BF = jnp.bfloat16
_F32 = jnp.float32


def _full_spec(shape):
    n = len(shape)
    return pl.BlockSpec(shape, lambda *a: (0,) * n)


def _s2d(x):
    B, H2, W2, C = x.shape
    x = x.reshape(B, H2 // 2, 2, W2 // 2, 2, C).transpose(0, 1, 3, 2, 4, 5)
    return x.reshape(B, H2 // 2, W2 // 2, 4 * C)


def _mm_relu(a, b, scale, shift, tm):
    M, K = a.shape
    K2, N = b.shape

    def body(a_ref, b_ref, s_ref, t_ref, o_ref):
        y = jnp.dot(a_ref[...], b_ref[...], preferred_element_type=_F32)
        y = y * s_ref[...] + t_ref[...]
        o_ref[...] = jnp.maximum(y, 0.0).astype(_BF)

    return pl.pallas_call(
        body,
        out_shape=jax.ShapeDtypeStruct((M, N), _BF),
        grid=(M // tm,),
        in_specs=[
            pl.BlockSpec((tm, K), lambda d: (d, 0)),
            _full_spec((K, N)),
            _full_spec((1, N)),
            _full_spec((1, N)),
        ],
        out_specs=pl.BlockSpec((tm, N), lambda d: (d, 0)),
        compiler_params=pltpu.CompilerParams(
            dimension_semantics=("parallel",)),
    )(a, b, scale.reshape(1, N), shift.reshape(1, N))


def _interleave2(y, D, H, W, Co):
    y = y.reshape(D, H, W, Co, 2, 2, 2)
    y = jnp.transpose(y, (0, 4, 1, 5, 2, 6, 3))
    return y.reshape(2 * D, 2 * H, 2 * W, Co)


def _ct2(y1, w, scale, shift):
    def body(x_ref, w_ref, s_ref, t_ref, o_ref):
        for h in range(2):
            a = x_ref[:, h * 64:(h + 1) * 64]
            y = jnp.dot(a, w_ref[...], preferred_element_type=_F32)
            y = jnp.maximum(y * s_ref[...] + t_ref[...], 0.0).astype(_BF)
            o_ref[h * 4096:(h + 1) * 4096, :] = y

    return pl.pallas_call(
        body,
        out_shape=jax.ShapeDtypeStruct((32768, 512), _BF),
        grid=(4,),
        in_specs=[
            pl.BlockSpec((4096, 128), lambda p: (0, p)),
            _full_spec((64, 512)),
            _full_spec((1, 512)),
            _full_spec((1, 512)),
        ],
        out_specs=pl.BlockSpec((8192, 512), lambda p: (p, 0)),
        compiler_params=pltpu.CompilerParams(
            dimension_semantics=("parallel",)),
    )(y1, w, scale.reshape(1, 512), shift.reshape(1, 512))


def _stem_pool_ps(y2r, w_all, scale, shift):
    def body(x_ref, w_ref, s_ref, t_ref, o_ref):
        xb = x_ref[...].reshape(4, 256, 256)
        P = {}
        for b1 in range(2):
            for c1 in range(2):
                plane = xb[b1 * 2 + c1]
                for b2 in range(2):
                    for c2 in range(2):
                        v = plane[:, (b2 * 2 + c2) * 64:(b2 * 2 + c2 + 1) * 64]
                        P[(2 * b1 + b2, 2 * c1 + c2)] = jnp.pad(
                            v.reshape(16, 16, 64), ((1, 1), (1, 1), (0, 0)))
        wins = []
        for g in range(2):
            for gw in range(2):
                parts = []
                for j7 in range(7):
                    o = 2 * g + j7 - 3
                    ph, dJ = o % 4, (o - o % 4) // 4
                    for l7 in range(7):
                        ow = 2 * gw + l7 - 3
                        pw, dK = ow % 4, (ow - ow % 4) // 4
                        parts.append(
                            P[(ph, pw)][1 + dJ:17 + dJ, 1 + dK:17 + dK, :])
                wins.append(jnp.concatenate(parts, axis=-1).reshape(256, 3136))
        big = jnp.concatenate(wins, axis=0)
        y = jnp.dot(big, w_ref[...], preferred_element_type=_F32)
        y = jnp.maximum(y * s_ref[...] + t_ref[...], 0.0).astype(_BF)
        y4p = jnp.pad(y.reshape(4, 16, 16, 64),
                      ((0, 0), (1, 0), (1, 0), (0, 0)))
        hs = ((1, 0), (0, 1), (1, 1))
        m = None
        for (g, r0) in hs:
            for (gw, s0) in hs:
                t_ = y4p[g * 2 + gw, r0:r0 + 16, s0:s0 + 16, :]
                m = t_ if m is None else jnp.maximum(m, t_)
        o_ref[...] = m.reshape(1, 16, 16, 64)

    return pl.pallas_call(
        body,
        out_shape=jax.ShapeDtypeStruct((64, 16, 16, 64), _BF),
        grid=(2, 16, 2),
        in_specs=[
            pl.BlockSpec((4, 1, 256, 256), lambda a1, i, a2: (a1, i, 0, a2)),
            _full_spec((3136, 64)),
            _full_spec((1, 64)),
            _full_spec((1, 64)),
        ],
        out_specs=pl.BlockSpec(
            (1, 16, 16, 64),
            lambda a1, i, a2: (4 * i + 2 * a1 + a2, 0, 0, 0)),
        compiler_params=pltpu.CompilerParams(
            dimension_semantics=("parallel", "parallel", "parallel")),
    )(y2r, w_all, scale.reshape(1, 64), shift.reshape(1, 64))


def _phase_major(wm, scale, shift, ci):
    w = wm.reshape(ci, 64, 8).transpose(0, 2, 1).reshape(ci, 512)
    s = scale.reshape(64, 8).transpose(1, 0).reshape(512)
    t = shift.reshape(64, 8).transpose(1, 0).reshape(512)
    return w, s, t


def _fused_chain(x, blks, *, t3, rd=16):
    D = 64
    n = len(blks)
    H, W = x.shape[1], x.shape[2]
    Co_last = blks[-1]["wc"].shape[1]

    def body(*refs):
        x_ref, o_ref = refs[0], refs[-1]
        idx = 1
        brefs = []
        for b in blks:
            cnt = 12 if b["br"] is not None else 9
            brefs.append(refs[idx:idx + cnt])
            idx += cnt
        d = pl.program_id(0)
        if t3:
            R = rd + 2 * n
            cur = x_ref[pl.ds(d * rd, R)]
        else:
            R = rd
            cur = x_ref[...]
        for m, (b, rs) in enumerate(zip(blks, brefs)):
            if b["br"] is not None:
                (wa_r, sa_r, ta_r, wb_r, sb_r, tb_r,
                 wc_r, sc_r, tc_r, wbr_r, sbr_r, tbr_r) = rs
            else:
                (wa_r, sa_r, ta_r, wb_r, sb_r, tb_r,
                 wc_r, sc_r, tc_r) = rs
                wbr_r = None
            Ci = cur.shape[3]
            I, Co = rs[6].shape
            if t3:
                Ro = R - 2
                xa = jnp.concatenate(
                    [cur[k:k + Ro] for k in range(3)], axis=-1)
                a_in = xa.reshape(Ro * H * W, 3 * Ci)
                xres = cur[1:Ro + 1]
            else:
                Ro = R
                a_in = cur.reshape(R * H * W, Ci)
                xres = cur
            h = jnp.dot(a_in, wa_r[...], preferred_element_type=_F32)
            h = jnp.maximum(h * sa_r[...] + ta_r[...], 0.0)
            hp = jnp.pad(h.astype(_BF).reshape(Ro, H, W, I),
                         ((0, 0), (1, 1), (1, 1), (0, 0)))
            acc = None
            for j in range(3):
                win = jnp.concatenate(
                    [hp[:, j:j + H, l:l + W, :] for l in range(3)], axis=-1)
                p = jnp.dot(win.reshape(Ro * H * W, 3 * I), wb_r[j],
                            preferred_element_type=_F32)
                acc = p if acc is None else acc + p
            h2 = jnp.maximum(acc * sb_r[...] + tb_r[...], 0.0).astype(_BF)
            y = jnp.dot(h2, wc_r[...], preferred_element_type=_F32)
            y = y * sc_r[...] + tc_r[...]
            if wbr_r is not None:
                rr = jnp.dot(xres.reshape(Ro * H * W, Ci), wbr_r[...],
                             preferred_element_type=_F32)
                rr = (rr * sbr_r[...] + tbr_r[...]).astype(_BF)
            else:
                rr = xres.reshape(Ro * H * W, Co)
            y = jnp.maximum(y + rr.astype(_F32), 0.0)
            y = y.reshape(Ro, H, W, Co).astype(_BF)
            if t3 and m < n - 1:
                gi = (d * rd - (n - m - 1)
                      + jax.lax.broadcasted_iota(jnp.int32,
                                                 (Ro, H, W, Co), 0))
                y = jnp.where((gi >= 0) & (gi < D), y, jnp.bfloat16(0))
            cur = y
            R = Ro
        o_ref[...] = cur

    if t3:
        x_spec = pl.BlockSpec((D + 2 * n, H, W, x.shape[3]),
                              lambda d: (0, 0, 0, 0))
    else:
        x_spec = pl.BlockSpec((rd, H, W, x.shape[3]), lambda d: (d, 0, 0, 0))
    in_specs = [x_spec]
    args = [x]
    for b in blks:
        I, Co = b["wc"].shape
        in_specs += [
            _full_spec(b["wa"].shape),
            _full_spec((1, I)), _full_spec((1, I)),
            _full_spec(b["wb3"].shape),
            _full_spec((1, I)), _full_spec((1, I)),
            _full_spec((I, Co)),
            _full_spec((1, Co)), _full_spec((1, Co)),
        ]
        args += [b["wa"], b["sa"].reshape(1, I), b["ta"].reshape(1, I),
                 b["wb3"], b["sb"].reshape(1, I), b["tb"].reshape(1, I),
                 b["wc"], b["sc"].reshape(1, Co), b["tc"].reshape(1, Co)]
        if b["br"] is not None:
            wbr, sbr, tbr = b["br"]
            in_specs += [_full_spec(wbr.shape),
                         _full_spec((1, Co)), _full_spec((1, Co))]
            args += [wbr, sbr.reshape(1, Co), tbr.reshape(1, Co)]

    return pl.pallas_call(
        body,
        out_shape=jax.ShapeDtypeStruct((D, H, W, Co_last), _BF),
        grid=(D // rd,),
        in_specs=in_specs,
        out_specs=pl.BlockSpec((rd, H, W, Co_last), lambda d: (d, 0, 0, 0)),
        compiler_params=pltpu.CompilerParams(
            dimension_semantics=("parallel",)),
    )(*args)


def _fused_block_s2(x, wa, sa, ta, wb3, sb, tb, wc, sc, tc, br,
                    *, t3, rd=16):
    D = 64
    H, W = x.shape[1], x.shape[2]
    Ci = x.shape[3]
    Hq, Wq = (H + 2) // 2, (W + 2) // 2
    I = wc.shape[0]
    Co = wc.shape[1]
    HO, WO = Hq - 1, Wq - 1
    wbr, sbr, tbr = br

    def _to_phases(v, R):
        vp = jnp.pad(v, ((0, 0), (1, 1), (1, 1), (0, 0)))
        t = vp.reshape(R, Hq, 2, W + 2, Ci)
        us = [t[:, :, a_].reshape(R, Hq, Wq, 2 * Ci) for a_ in range(2)]
        return jnp.concatenate(us, axis=-1)

    def body(x_ref, wa_ref, sa_ref, ta_ref, wb_ref, sb_ref, tb_ref,
             wc_ref, sc_ref, tc_ref, wbr_ref, sbr_ref, tbr_ref, o_ref):
        if t3:
            d = pl.program_id(0)
            xw = x_ref[pl.ds(d * rd, rd + 2)]
            xs = _to_phases(xw, rd + 2)

            def a_in(p):
                return jnp.concatenate(
                    [xs[k:k + rd, :, :, p * Ci:(p + 1) * Ci]
                     for k in range(3)], axis=-1).reshape(rd * Hq * Wq, 3 * Ci)
            xmid = xs[1:rd + 1]
        else:
            xs = _to_phases(x_ref[...], rd)

            def a_in(p):
                return xs[:, :, :, p * Ci:(p + 1) * Ci].reshape(
                    rd * Hq * Wq, Ci)
            xmid = xs

        qrow = jax.lax.broadcasted_iota(jnp.int32, (rd, Hq, Wq, I), 1)
        qcol = jax.lax.broadcasted_iota(jnp.int32, (rd, Hq, Wq, I), 2)
        hph = []
        for a in range(2):
            for b in range(2):
                hv = jnp.dot(a_in(2 * a + b), wa_ref[...],
                             preferred_element_type=_F32)
                hv = jnp.maximum(hv * sa_ref[...] + ta_ref[...], 0.0)
                hv = hv.astype(_BF).reshape(rd, Hq, Wq, I)
                rok = (qrow > 0) if a == 0 else (qrow < Hq - 1)
                cok = (qcol > 0) if b == 0 else (qcol < Wq - 1)
                hph.append(jnp.where(rok & cok, hv, jnp.bfloat16(0)))

        jmap = ((0, 0), (0, 1), (1, 0))
        acc = None
        for j in range(3):
            J, a = jmap[j]
            parts = []
            for l in range(3):
                Lw, b = jmap[l]
                parts.append(hph[2 * a + b][:, J:J + HO, Lw:Lw + WO, :])
            win = jnp.concatenate(parts, axis=-1)
            p = jnp.dot(win.reshape(rd * HO * WO, 3 * I), wb_ref[j],
                        preferred_element_type=_F32)
            acc = p if acc is None else acc + p
        h2 = jnp.maximum(acc * sb_ref[...] + tb_ref[...], 0.0).astype(_BF)

        y = jnp.dot(h2, wc_ref[...], preferred_element_type=_F32)
        y = y * sc_ref[...] + tc_ref[...]
        xr = xmid[:, 0:HO, 0:WO, 3 * Ci:4 * Ci]
        rr = jnp.dot(xr.reshape(rd * HO * WO, Ci), wbr_ref[...],
                     preferred_element_type=_F32)
        rr = (rr * sbr_ref[...] + tbr_ref[...]).astype(_BF)
        y = jnp.maximum(y + rr.astype(_F32), 0.0)
        o_ref[...] = y.reshape(rd, HO, WO, Co).astype(_BF)

    if t3:
        x_spec = pl.BlockSpec((D + 2, H, W, Ci), lambda d: (0, 0, 0, 0))
    else:
        x_spec = pl.BlockSpec((rd, H, W, Ci), lambda d: (d, 0, 0, 0))

    return pl.pallas_call(
        body,
        out_shape=jax.ShapeDtypeStruct((D, HO, WO, Co), _BF),
        grid=(D // rd,),
        in_specs=[
            x_spec,
            _full_spec(wa.shape),
            _full_spec((1, I)), _full_spec((1, I)),
            _full_spec(wb3.shape),
            _full_spec((1, I)), _full_spec((1, I)),
            _full_spec((I, Co)),
            _full_spec((1, Co)), _full_spec((1, Co)),
            _full_spec((Ci, Co)),
            _full_spec((1, Co)), _full_spec((1, Co)),
        ],
        out_specs=pl.BlockSpec((rd, HO, WO, Co), lambda d: (d, 0, 0, 0)),
        compiler_params=pltpu.CompilerParams(
            dimension_semantics=("parallel",)),
    )(x, wa, sa.reshape(1, I), ta.reshape(1, I),
      wb3, sb.reshape(1, I), tb.reshape(1, I),
      wc, sc.reshape(1, Co), tc.reshape(1, Co),
      wbr, sbr.reshape(1, Co), tbr.reshape(1, Co))


def _head(xf, w1, s1, t1, w2, s2, t2):
    def body(x_ref, w1_ref, s1_ref, t1_ref, w2_ref, s2_ref, t2_ref, o_ref):
        m = jnp.sum(x_ref[...].astype(_F32), axis=0, keepdims=True)
        m = m * (1.0 / 256.0)
        f1 = jnp.dot(m.astype(_BF), w1_ref[...], preferred_element_type=_F32)
        f1 = f1 * s1_ref[...] + t1_ref[...]
        f2 = jnp.dot(f1.astype(_BF), w2_ref[...], preferred_element_type=_F32)
        f2 = jax.nn.sigmoid(f2 * s2_ref[...] + t2_ref[...])
        o_ref[...] = jnp.broadcast_to(f2, (8, 128))

    return pl.pallas_call(
        body,
        out_shape=jax.ShapeDtypeStruct((8, 128), _F32),
        grid=(1,),
        in_specs=[_full_spec((256, 2048)),
                  _full_spec((2048, 128)),
                  _full_spec((1, 128)), _full_spec((1, 128)),
                  _full_spec((128, 128)),
                  _full_spec((1, 128)), _full_spec((1, 128))],
        out_specs=_full_spec((8, 128)),
        compiler_params=pltpu.CompilerParams(
            dimension_semantics=("arbitrary",)),
    )(xf, w1, s1, t1, w2, s2, t2)


def _unpack_s2d_taps(wb, k):
    k2 = (k + 1) // 2
    T, C4, Co = wb.shape
    Ci = C4 // 4
    w = wb.reshape(k2, k2, 2, 2, Ci, Co).transpose(0, 2, 1, 3, 4, 5)
    w = w.reshape(2 * k2, 2 * k2, Ci, Co)[:k, :k]
    return w.reshape(k, k * Ci, Co)


_T3 = frozenset(range(7, 16))
_STRIDED = frozenset((3, 7, 13))
_BRANCH = frozenset((0, 3, 7, 13))


def kernel(ct_input, ct1_wm, ct1_scale, ct1_shift, ct2_wm, ct2_scale, ct2_shift, stem_w, stem_scale, stem_shift, b0_a_w, b0_a_scale, b0_a_shift, b0_b_w, b0_b_scale, b0_b_shift, b0_c_w, b0_c_scale, b0_c_shift, b0_br_w, b0_br_scale, b0_br_shift, b1_a_w, b1_a_scale, b1_a_shift, b1_b_w, b1_b_scale, b1_b_shift, b1_c_w, b1_c_scale, b1_c_shift, b2_a_w, b2_a_scale, b2_a_shift, b2_b_w, b2_b_scale, b2_b_shift, b2_c_w, b2_c_scale, b2_c_shift, b3_a_w, b3_a_scale, b3_a_shift, b3_b_w, b3_b_scale, b3_b_shift, b3_c_w, b3_c_scale, b3_c_shift, b3_br_w, b3_br_scale, b3_br_shift, b4_a_w, b4_a_scale, b4_a_shift, b4_b_w, b4_b_scale, b4_b_shift, b4_c_w, b4_c_scale, b4_c_shift, b5_a_w, b5_a_scale, b5_a_shift, b5_b_w, b5_b_scale, b5_b_shift, b5_c_w, b5_c_scale, b5_c_shift, b6_a_w, b6_a_scale, b6_a_shift, b6_b_w, b6_b_scale, b6_b_shift, b6_c_w, b6_c_scale, b6_c_shift, b7_a_w, b7_a_scale, b7_a_shift, b7_b_w, b7_b_scale, b7_b_shift, b7_c_w, b7_c_scale, b7_c_shift, b7_br_w, b7_br_scale, b7_br_shift, b8_a_w, b8_a_scale, b8_a_shift, b8_b_w, b8_b_scale, b8_b_shift, b8_c_w, b8_c_scale, b8_c_shift, b9_a_w, b9_a_scale, b9_a_shift, b9_b_w, b9_b_scale, b9_b_shift, b9_c_w, b9_c_scale, b9_c_shift, b10_a_w, b10_a_scale, b10_a_shift, b10_b_w, b10_b_scale, b10_b_shift, b10_c_w, b10_c_scale, b10_c_shift, b11_a_w, b11_a_scale, b11_a_shift, b11_b_w, b11_b_scale, b11_b_shift, b11_c_w, b11_c_scale, b11_c_shift, b12_a_w, b12_a_scale, b12_a_shift, b12_b_w, b12_b_scale, b12_b_shift, b12_c_w, b12_c_scale, b12_c_shift, b13_a_w, b13_a_scale, b13_a_shift, b13_b_w, b13_b_scale, b13_b_shift, b13_c_w, b13_c_scale, b13_c_shift, b13_br_w, b13_br_scale, b13_br_shift, b14_a_w, b14_a_scale, b14_a_shift, b14_b_w, b14_b_scale, b14_b_shift, b14_c_w, b14_c_scale, b14_c_shift, b15_a_w, b15_a_scale, b15_a_shift, b15_b_w, b15_b_scale, b15_b_shift, b15_c_w, b15_c_scale, b15_c_shift, head_proj_w, head_proj_scale, head_proj_shift, final_w, final_scale, final_shift):
    L = locals()

    x = jnp.transpose(ct_input.astype(_F32), (0, 2, 3, 4, 1))
    a = jnp.pad(x.reshape(16 * 16 * 16, 2).astype(_BF), ((0, 0), (0, 6)))
    w1m, s1m, t1m = _phase_major(ct1_wm, ct1_scale, ct1_shift, 2)
    y1 = _mm_relu(a, jnp.pad(w1m, ((0, 6), (0, 0))), s1m, t1m, tm=2048)
    w2m, s2m, t2m = _phase_major(ct2_wm, ct2_scale, ct2_shift, 64)
    y2 = _ct2(y1, w2m, s2m, t2m)
    w49 = stem_w.reshape(4, 4, 2, 2, 64, 64).transpose(0, 2, 1, 3, 4, 5)
    w49 = w49.reshape(8, 8, 64, 64)[:7, :7].reshape(49 * 64, 64)
    x = _stem_pool_ps(y2.reshape(8, 16, 256, 512), w49,
                      stem_scale, stem_shift)

    def bp(i):
        t3 = i in _T3
        wa = L[f"b{i}_a_w"]
        if t3:
            wa = wa.reshape(3 * wa.shape[1], wa.shape[2])
        wb = L[f"b{i}_b_w"]
        if i in _STRIDED:
            wb3 = _unpack_s2d_taps(wb, 3)
        else:
            wb3 = wb.reshape(3, 3 * wb.shape[1], wb.shape[2])
        br = None
        if i in _BRANCH:
            br = (L[f"b{i}_br_w"], L[f"b{i}_br_scale"], L[f"b{i}_br_shift"])
        return dict(
            wa=wa, sa=L[f"b{i}_a_scale"], ta=L[f"b{i}_a_shift"],
            wb3=wb3, sb=L[f"b{i}_b_scale"], tb=L[f"b{i}_b_shift"],
            wc=L[f"b{i}_c_w"], sc=L[f"b{i}_c_scale"], tc=L[f"b{i}_c_shift"],
            br=br)

    def strided(x, p, t3):
        if t3:
            x = jnp.pad(x, ((1, 1), (0, 0), (0, 0), (0, 0)))
        return _fused_block_s2(x, p["wa"], p["sa"], p["ta"], p["wb3"],
                               p["sb"], p["tb"], p["wc"], p["sc"], p["tc"],
                               p["br"], t3=t3)

    x = _fused_chain(x, [bp(0), bp(1), bp(2)], t3=False, rd=8)
    x = strided(x, bp(3), False)
    x = _fused_chain(x, [bp(4), bp(5), bp(6)], t3=False, rd=8)
    x = strided(x, bp(7), True)
    x = jnp.pad(x, ((5, 5), (0, 0), (0, 0), (0, 0)))
    x = _fused_chain(x, [bp(8), bp(9), bp(10), bp(11), bp(12)], t3=True)
    x = strided(x, bp(13), True)
    x = jnp.pad(x, ((2, 2), (0, 0), (0, 0), (0, 0)))
    x = _fused_chain(x, [bp(14), bp(15)], t3=True)

    xf = x.reshape(256, 2048)
    w1 = jnp.pad(head_proj_w, ((0, 0), (0, 67)))
    s1 = jnp.pad(head_proj_scale, (0, 67)).reshape(1, 128)
    t1 = jnp.pad(head_proj_shift, (0, 67)).reshape(1, 128)
    w2 = jnp.pad(final_w, ((0, 67), (0, 124)))
    s2 = jnp.pad(final_scale, (0, 124)).reshape(1, 128)
    t2 = jnp.pad(final_shift, (0, 124)).reshape(1, 128)
    out = _head(xf, w1, s1, t1, w2, s2, t2)
    return out[0:1, 0:4]


# speedup vs baseline: 1.0395x; 1.0395x over previous
import jax
import jax.numpy as jnp
from jax.experimental import pallas as pl
from jax.experimental.pallas import tpu as pltpu

_BF = jnp.bfloat16
_F32 = jnp.float32


def _full_spec(shape):
    n = len(shape)
    return pl.BlockSpec(shape, lambda *a: (0,) * n)


def _s2d(x):
    B, H2, W2, C = x.shape
    x = x.reshape(B, H2 // 2, 2, W2 // 2, 2, C).transpose(0, 1, 3, 2, 4, 5)
    return x.reshape(B, H2 // 2, W2 // 2, 4 * C)


def _mm_relu(a, b, scale, shift, tm):
    M, K = a.shape
    K2, N = b.shape

    def body(a_ref, b_ref, s_ref, t_ref, o_ref):
        y = jnp.dot(a_ref[...], b_ref[...], preferred_element_type=_F32)
        y = y * s_ref[...] + t_ref[...]
        o_ref[...] = jnp.maximum(y, 0.0).astype(_BF)

    return pl.pallas_call(
        body,
        out_shape=jax.ShapeDtypeStruct((M, N), _BF),
        grid=(M // tm,),
        in_specs=[
            pl.BlockSpec((tm, K), lambda d: (d, 0)),
            _full_spec((K, N)),
            _full_spec((1, N)),
            _full_spec((1, N)),
        ],
        out_specs=pl.BlockSpec((tm, N), lambda d: (d, 0)),
        compiler_params=pltpu.CompilerParams(
            dimension_semantics=("parallel",)),
    )(a, b, scale.reshape(1, N), shift.reshape(1, N))


def _interleave2(y, D, H, W, Co):
    y = y.reshape(D, H, W, Co, 2, 2, 2)
    y = jnp.transpose(y, (0, 4, 1, 5, 2, 6, 3))
    return y.reshape(2 * D, 2 * H, 2 * W, Co)


def _ct2(y1, w, scale, shift):
    def body(x_ref, w_ref, s_ref, t_ref, o_ref):
        for h in range(2):
            a = x_ref[:, h * 64:(h + 1) * 64]
            y = jnp.dot(a, w_ref[...], preferred_element_type=_F32)
            y = jnp.maximum(y * s_ref[...] + t_ref[...], 0.0).astype(_BF)
            o_ref[h * 4096:(h + 1) * 4096, :] = y

    return pl.pallas_call(
        body,
        out_shape=jax.ShapeDtypeStruct((32768, 512), _BF),
        grid=(4,),
        in_specs=[
            pl.BlockSpec((4096, 128), lambda p: (0, p)),
            _full_spec((64, 512)),
            _full_spec((1, 512)),
            _full_spec((1, 512)),
        ],
        out_specs=pl.BlockSpec((8192, 512), lambda p: (p, 0)),
        compiler_params=pltpu.CompilerParams(
            dimension_semantics=("parallel",)),
    )(y1, w, scale.reshape(1, 512), shift.reshape(1, 512))


def _stem_pool_ps(y2r, w_all, scale, shift):
    def body(x_ref, w_ref, s_ref, t_ref, o_ref):
        xb = x_ref[...].reshape(4, 256, 256)
        P = {}
        for b1 in range(2):
            for c1 in range(2):
                plane = xb[b1 * 2 + c1]
                for b2 in range(2):
                    for c2 in range(2):
                        v = plane[:, (b2 * 2 + c2) * 64:(b2 * 2 + c2 + 1) * 64]
                        P[(2 * b1 + b2, 2 * c1 + c2)] = jnp.pad(
                            v.reshape(16, 16, 64), ((1, 1), (1, 1), (0, 0)))
        wins = []
        for g in range(2):
            for gw in range(2):
                parts = []
                for j7 in range(7):
                    o = 2 * g + j7 - 3
                    ph, dJ = o % 4, (o - o % 4) // 4
                    for l7 in range(7):
                        ow = 2 * gw + l7 - 3
                        pw, dK = ow % 4, (ow - ow % 4) // 4
                        parts.append(
                            P[(ph, pw)][1 + dJ:17 + dJ, 1 + dK:17 + dK, :])
                wins.append(jnp.concatenate(parts, axis=-1).reshape(256, 3136))
        big = jnp.concatenate(wins, axis=0)
        y = jnp.dot(big, w_ref[...], preferred_element_type=_F32)
        y = jnp.maximum(y * s_ref[...] + t_ref[...], 0.0).astype(_BF)
        y4p = jnp.pad(y.reshape(4, 16, 16, 64),
                      ((0, 0), (1, 0), (1, 0), (0, 0)))
        hs = ((1, 0), (0, 1), (1, 1))
        m = None
        for (g, r0) in hs:
            for (gw, s0) in hs:
                t_ = y4p[g * 2 + gw, r0:r0 + 16, s0:s0 + 16, :]
                m = t_ if m is None else jnp.maximum(m, t_)
        o_ref[...] = m.reshape(1, 16, 16, 64)

    return pl.pallas_call(
        body,
        out_shape=jax.ShapeDtypeStruct((64, 16, 16, 64), _BF),
        grid=(2, 16, 2),
        in_specs=[
            pl.BlockSpec((4, 1, 256, 256), lambda a1, i, a2: (a1, i, 0, a2)),
            _full_spec((3136, 64)),
            _full_spec((1, 64)),
            _full_spec((1, 64)),
        ],
        out_specs=pl.BlockSpec(
            (1, 16, 16, 64),
            lambda a1, i, a2: (4 * i + 2 * a1 + a2, 0, 0, 0)),
        compiler_params=pltpu.CompilerParams(
            dimension_semantics=("parallel", "parallel", "parallel")),
    )(y2r, w_all, scale.reshape(1, 64), shift.reshape(1, 64))


def _phase_major(wm, scale, shift, ci):
    w = wm.reshape(ci, 64, 8).transpose(0, 2, 1).reshape(ci, 512)
    s = scale.reshape(64, 8).transpose(1, 0).reshape(512)
    t = shift.reshape(64, 8).transpose(1, 0).reshape(512)
    return w, s, t


def _fused_chain(x, blks, *, t3, rd=16):
    D = 64
    n = len(blks)
    H, W = x.shape[1], x.shape[2]
    Co_last = blks[-1]["wc"].shape[1]

    def body(*refs):
        x_ref, o_ref = refs[0], refs[-1]
        idx = 1
        brefs = []
        for b in blks:
            cnt = 12 if b["br"] is not None else 9
            brefs.append(refs[idx:idx + cnt])
            idx += cnt
        d = pl.program_id(0)
        if t3:
            R = rd + 2 * n
            cur = x_ref[pl.ds(d * rd, R)]
        else:
            R = rd
            cur = x_ref[...]
        for m, (b, rs) in enumerate(zip(blks, brefs)):
            if b["br"] is not None:
                (wa_r, sa_r, ta_r, wb_r, sb_r, tb_r,
                 wc_r, sc_r, tc_r, wbr_r, sbr_r, tbr_r) = rs
            else:
                (wa_r, sa_r, ta_r, wb_r, sb_r, tb_r,
                 wc_r, sc_r, tc_r) = rs
                wbr_r = None
            Ci = cur.shape[3]
            I, Co = rs[6].shape
            if t3:
                Ro = R - 2
                xa = jnp.concatenate(
                    [cur[k:k + Ro] for k in range(3)], axis=-1)
                a_in = xa.reshape(Ro * H * W, 3 * Ci)
                xres = cur[1:Ro + 1]
            else:
                Ro = R
                a_in = cur.reshape(R * H * W, Ci)
                xres = cur
            h = jnp.dot(a_in, wa_r[...], preferred_element_type=_F32)
            h = jnp.maximum(h * sa_r[...] + ta_r[...], 0.0)
            hp = jnp.pad(h.astype(_BF).reshape(Ro, H, W, I),
                         ((0, 0), (1, 1), (1, 1), (0, 0)))
            acc = None
            for j in range(3):
                win = jnp.concatenate(
                    [hp[:, j:j + H, l:l + W, :] for l in range(3)], axis=-1)
                p = jnp.dot(win.reshape(Ro * H * W, 3 * I), wb_r[j],
                            preferred_element_type=_F32)
                acc = p if acc is None else acc + p
            h2 = jnp.maximum(acc * sb_r[...] + tb_r[...], 0.0).astype(_BF)
            y = jnp.dot(h2, wc_r[...], preferred_element_type=_F32)
            y = y * sc_r[...] + tc_r[...]
            if wbr_r is not None:
                rr = jnp.dot(xres.reshape(Ro * H * W, Ci), wbr_r[...],
                             preferred_element_type=_F32)
                rr = (rr * sbr_r[...] + tbr_r[...]).astype(_BF)
            else:
                rr = xres.reshape(Ro * H * W, Co)
            y = jnp.maximum(y + rr.astype(_F32), 0.0)
            y = y.reshape(Ro, H, W, Co).astype(_BF)
            if t3 and m < n - 1:
                gi = (d * rd - (n - m - 1)
                      + jax.lax.broadcasted_iota(jnp.int32,
                                                 (Ro, H, W, Co), 0))
                y = jnp.where((gi >= 0) & (gi < D), y, jnp.bfloat16(0))
            cur = y
            R = Ro
        o_ref[...] = cur

    if t3:
        x_spec = pl.BlockSpec((D + 2 * n, H, W, x.shape[3]),
                              lambda d: (0, 0, 0, 0))
    else:
        x_spec = pl.BlockSpec((rd, H, W, x.shape[3]), lambda d: (d, 0, 0, 0))
    in_specs = [x_spec]
    args = [x]
    for b in blks:
        I, Co = b["wc"].shape
        in_specs += [
            _full_spec(b["wa"].shape),
            _full_spec((1, I)), _full_spec((1, I)),
            _full_spec(b["wb3"].shape),
            _full_spec((1, I)), _full_spec((1, I)),
            _full_spec((I, Co)),
            _full_spec((1, Co)), _full_spec((1, Co)),
        ]
        args += [b["wa"], b["sa"].reshape(1, I), b["ta"].reshape(1, I),
                 b["wb3"], b["sb"].reshape(1, I), b["tb"].reshape(1, I),
                 b["wc"], b["sc"].reshape(1, Co), b["tc"].reshape(1, Co)]
        if b["br"] is not None:
            wbr, sbr, tbr = b["br"]
            in_specs += [_full_spec(wbr.shape),
                         _full_spec((1, Co)), _full_spec((1, Co))]
            args += [wbr, sbr.reshape(1, Co), tbr.reshape(1, Co)]

    return pl.pallas_call(
        body,
        out_shape=jax.ShapeDtypeStruct((D, H, W, Co_last), _BF),
        grid=(D // rd,),
        in_specs=in_specs,
        out_specs=pl.BlockSpec((rd, H, W, Co_last), lambda d: (d, 0, 0, 0)),
        compiler_params=pltpu.CompilerParams(
            dimension_semantics=("parallel",)),
    )(*args)


def _fused_block_s2(x, wa, sa, ta, wb3, sb, tb, wc, sc, tc, br,
                    *, t3, rd=16):
    D = 64
    H, W = x.shape[1], x.shape[2]
    Ci = x.shape[3]
    Hq, Wq = (H + 2) // 2, (W + 2) // 2
    I = wc.shape[0]
    Co = wc.shape[1]
    HO, WO = Hq - 1, Wq - 1
    wbr, sbr, tbr = br

    def _to_phases(v, R):
        vp = jnp.pad(v, ((0, 0), (1, 1), (1, 1), (0, 0)))
        t = vp.reshape(R, Hq, 2, W + 2, Ci)
        us = [t[:, :, a_].reshape(R, Hq, Wq, 2 * Ci) for a_ in range(2)]
        return jnp.concatenate(us, axis=-1)

    def body(x_ref, wa_ref, sa_ref, ta_ref, wb_ref, sb_ref, tb_ref,
             wc_ref, sc_ref, tc_ref, wbr_ref, sbr_ref, tbr_ref, o_ref):
        if t3:
            d = pl.program_id(0)
            xw = x_ref[pl.ds(d * rd, rd + 2)]
            xs = _to_phases(xw, rd + 2)

            def a_in(p):
                return jnp.concatenate(
                    [xs[k:k + rd, :, :, p * Ci:(p + 1) * Ci]
                     for k in range(3)], axis=-1).reshape(rd * Hq * Wq, 3 * Ci)
            xmid = xs[1:rd + 1]
        else:
            xs = _to_phases(x_ref[...], rd)

            def a_in(p):
                return xs[:, :, :, p * Ci:(p + 1) * Ci].reshape(
                    rd * Hq * Wq, Ci)
            xmid = xs

        qrow = jax.lax.broadcasted_iota(jnp.int32, (rd, Hq, Wq, I), 1)
        qcol = jax.lax.broadcasted_iota(jnp.int32, (rd, Hq, Wq, I), 2)
        hph = []
        for a in range(2):
            for b in range(2):
                hv = jnp.dot(a_in(2 * a + b), wa_ref[...],
                             preferred_element_type=_F32)
                hv = jnp.maximum(hv * sa_ref[...] + ta_ref[...], 0.0)
                hv = hv.astype(_BF).reshape(rd, Hq, Wq, I)
                rok = (qrow > 0) if a == 0 else (qrow < Hq - 1)
                cok = (qcol > 0) if b == 0 else (qcol < Wq - 1)
                hph.append(jnp.where(rok & cok, hv, jnp.bfloat16(0)))

        jmap = ((0, 0), (0, 1), (1, 0))
        acc = None
        for j in range(3):
            J, a = jmap[j]
            parts = []
            for l in range(3):
                Lw, b = jmap[l]
                parts.append(hph[2 * a + b][:, J:J + HO, Lw:Lw + WO, :])
            win = jnp.concatenate(parts, axis=-1)
            p = jnp.dot(win.reshape(rd * HO * WO, 3 * I), wb_ref[j],
                        preferred_element_type=_F32)
            acc = p if acc is None else acc + p
        h2 = jnp.maximum(acc * sb_ref[...] + tb_ref[...], 0.0).astype(_BF)

        y = jnp.dot(h2, wc_ref[...], preferred_element_type=_F32)
        y = y * sc_ref[...] + tc_ref[...]
        xr = xmid[:, 0:HO, 0:WO, 3 * Ci:4 * Ci]
        rr = jnp.dot(xr.reshape(rd * HO * WO, Ci), wbr_ref[...],
                     preferred_element_type=_F32)
        rr = (rr * sbr_ref[...] + tbr_ref[...]).astype(_BF)
        y = jnp.maximum(y + rr.astype(_F32), 0.0)
        o_ref[...] = y.reshape(rd, HO, WO, Co).astype(_BF)

    if t3:
        x_spec = pl.BlockSpec((D + 2, H, W, Ci), lambda d: (0, 0, 0, 0))
    else:
        x_spec = pl.BlockSpec((rd, H, W, Ci), lambda d: (d, 0, 0, 0))

    return pl.pallas_call(
        body,
        out_shape=jax.ShapeDtypeStruct((D, HO, WO, Co), _BF),
        grid=(D // rd,),
        in_specs=[
            x_spec,
            _full_spec(wa.shape),
            _full_spec((1, I)), _full_spec((1, I)),
            _full_spec(wb3.shape),
            _full_spec((1, I)), _full_spec((1, I)),
            _full_spec((I, Co)),
            _full_spec((1, Co)), _full_spec((1, Co)),
            _full_spec((Ci, Co)),
            _full_spec((1, Co)), _full_spec((1, Co)),
        ],
        out_specs=pl.BlockSpec((rd, HO, WO, Co), lambda d: (d, 0, 0, 0)),
        compiler_params=pltpu.CompilerParams(
            dimension_semantics=("parallel",)),
    )(x, wa, sa.reshape(1, I), ta.reshape(1, I),
      wb3, sb.reshape(1, I), tb.reshape(1, I),
      wc, sc.reshape(1, Co), tc.reshape(1, Co),
      wbr, sbr.reshape(1, Co), tbr.reshape(1, Co))


def _head(xf, w1, s1, t1, w2, s2, t2):
    def body(x_ref, w1_ref, s1_ref, t1_ref, w2_ref, s2_ref, t2_ref, o_ref):
        m = jnp.sum(x_ref[...].astype(_F32), axis=0, keepdims=True)
        m = m * (1.0 / 256.0)
        f1 = jnp.dot(m.astype(_BF), w1_ref[...], preferred_element_type=_F32)
        f1 = f1 * s1_ref[...] + t1_ref[...]
        f2 = jnp.dot(f1.astype(_BF), w2_ref[...], preferred_element_type=_F32)
        f2 = jax.nn.sigmoid(f2 * s2_ref[...] + t2_ref[...])
        o_ref[...] = jnp.broadcast_to(f2, (8, 128))

    return pl.pallas_call(
        body,
        out_shape=jax.ShapeDtypeStruct((8, 128), _F32),
        grid=(1,),
        in_specs=[_full_spec((256, 2048)),
                  _full_spec((2048, 128)),
                  _full_spec((1, 128)), _full_spec((1, 128)),
                  _full_spec((128, 128)),
                  _full_spec((1, 128)), _full_spec((1, 128))],
        out_specs=_full_spec((8, 128)),
        compiler_params=pltpu.CompilerParams(
            dimension_semantics=("arbitrary",)),
    )(xf, w1, s1, t1, w2, s2, t2)


def _unpack_s2d_taps(wb, k):
    k2 = (k + 1) // 2
    T, C4, Co = wb.shape
    Ci = C4 // 4
    w = wb.reshape(k2, k2, 2, 2, Ci, Co).transpose(0, 2, 1, 3, 4, 5)
    w = w.reshape(2 * k2, 2 * k2, Ci, Co)[:k, :k]
    return w.reshape(k, k * Ci, Co)


_T3 = frozenset(range(7, 16))
_STRIDED = frozenset((3, 7, 13))
_BRANCH = frozenset((0, 3, 7, 13))


def kernel(ct_input, ct1_wm, ct1_scale, ct1_shift, ct2_wm, ct2_scale, ct2_shift, stem_w, stem_scale, stem_shift, b0_a_w, b0_a_scale, b0_a_shift, b0_b_w, b0_b_scale, b0_b_shift, b0_c_w, b0_c_scale, b0_c_shift, b0_br_w, b0_br_scale, b0_br_shift, b1_a_w, b1_a_scale, b1_a_shift, b1_b_w, b1_b_scale, b1_b_shift, b1_c_w, b1_c_scale, b1_c_shift, b2_a_w, b2_a_scale, b2_a_shift, b2_b_w, b2_b_scale, b2_b_shift, b2_c_w, b2_c_scale, b2_c_shift, b3_a_w, b3_a_scale, b3_a_shift, b3_b_w, b3_b_scale, b3_b_shift, b3_c_w, b3_c_scale, b3_c_shift, b3_br_w, b3_br_scale, b3_br_shift, b4_a_w, b4_a_scale, b4_a_shift, b4_b_w, b4_b_scale, b4_b_shift, b4_c_w, b4_c_scale, b4_c_shift, b5_a_w, b5_a_scale, b5_a_shift, b5_b_w, b5_b_scale, b5_b_shift, b5_c_w, b5_c_scale, b5_c_shift, b6_a_w, b6_a_scale, b6_a_shift, b6_b_w, b6_b_scale, b6_b_shift, b6_c_w, b6_c_scale, b6_c_shift, b7_a_w, b7_a_scale, b7_a_shift, b7_b_w, b7_b_scale, b7_b_shift, b7_c_w, b7_c_scale, b7_c_shift, b7_br_w, b7_br_scale, b7_br_shift, b8_a_w, b8_a_scale, b8_a_shift, b8_b_w, b8_b_scale, b8_b_shift, b8_c_w, b8_c_scale, b8_c_shift, b9_a_w, b9_a_scale, b9_a_shift, b9_b_w, b9_b_scale, b9_b_shift, b9_c_w, b9_c_scale, b9_c_shift, b10_a_w, b10_a_scale, b10_a_shift, b10_b_w, b10_b_scale, b10_b_shift, b10_c_w, b10_c_scale, b10_c_shift, b11_a_w, b11_a_scale, b11_a_shift, b11_b_w, b11_b_scale, b11_b_shift, b11_c_w, b11_c_scale, b11_c_shift, b12_a_w, b12_a_scale, b12_a_shift, b12_b_w, b12_b_scale, b12_b_shift, b12_c_w, b12_c_scale, b12_c_shift, b13_a_w, b13_a_scale, b13_a_shift, b13_b_w, b13_b_scale, b13_b_shift, b13_c_w, b13_c_scale, b13_c_shift, b13_br_w, b13_br_scale, b13_br_shift, b14_a_w, b14_a_scale, b14_a_shift, b14_b_w, b14_b_scale, b14_b_shift, b14_c_w, b14_c_scale, b14_c_shift, b15_a_w, b15_a_scale, b15_a_shift, b15_b_w, b15_b_scale, b15_b_shift, b15_c_w, b15_c_scale, b15_c_shift, head_proj_w, head_proj_scale, head_proj_shift, final_w, final_scale, final_shift):
    L = locals()

    x = jnp.transpose(ct_input.astype(_F32), (0, 2, 3, 4, 1))
    a = jnp.pad(x.reshape(16 * 16 * 16, 2).astype(_BF), ((0, 0), (0, 6)))
    w1m, s1m, t1m = _phase_major(ct1_wm, ct1_scale, ct1_shift, 2)
    y1 = _mm_relu(a, jnp.pad(w1m, ((0, 6), (0, 0))), s1m, t1m, tm=2048)
    w2m, s2m, t2m = _phase_major(ct2_wm, ct2_scale, ct2_shift, 64)
    y2 = _ct2(y1, w2m, s2m, t2m)
    w49 = stem_w.reshape(4, 4, 2, 2, 64, 64).transpose(0, 2, 1, 3, 4, 5)
    w49 = w49.reshape(8, 8, 64, 64)[:7, :7].reshape(49 * 64, 64)
    x = _stem_pool_ps(y2.reshape(8, 16, 256, 512), w49,
                      stem_scale, stem_shift)

    def bp(i):
        t3 = i in _T3
        wa = L[f"b{i}_a_w"]
        if t3:
            wa = wa.reshape(3 * wa.shape[1], wa.shape[2])
        wb = L[f"b{i}_b_w"]
        if i in _STRIDED:
            wb3 = _unpack_s2d_taps(wb, 3)
        else:
            wb3 = wb.reshape(3, 3 * wb.shape[1], wb.shape[2])
        br = None
        if i in _BRANCH:
            br = (L[f"b{i}_br_w"], L[f"b{i}_br_scale"], L[f"b{i}_br_shift"])
        return dict(
            wa=wa, sa=L[f"b{i}_a_scale"], ta=L[f"b{i}_a_shift"],
            wb3=wb3, sb=L[f"b{i}_b_scale"], tb=L[f"b{i}_b_shift"],
            wc=L[f"b{i}_c_w"], sc=L[f"b{i}_c_scale"], tc=L[f"b{i}_c_shift"],
            br=br)

    def strided(x, p, t3):
        if t3:
            x = jnp.pad(x, ((1, 1), (0, 0), (0, 0), (0, 0)))
        return _fused_block_s2(x, p["wa"], p["sa"], p["ta"], p["wb3"],
                               p["sb"], p["tb"], p["wc"], p["sc"], p["tc"],
                               p["br"], t3=t3)

    x = _fused_chain(x, [bp(0), bp(1), bp(2)], t3=False)
    x = strided(x, bp(3), False)
    x = _fused_chain(x, [bp(4), bp(5), bp(6)], t3=False)
    x = strided(x, bp(7), True)
    x = jnp.pad(x, ((5, 5), (0, 0), (0, 0), (0, 0)))
    x = _fused_chain(x, [bp(8), bp(9), bp(10), bp(11), bp(12)], t3=True)
    x = strided(x, bp(13), True)
    x = jnp.pad(x, ((2, 2), (0, 0), (0, 0), (0, 0)))
    x = _fused_chain(x, [bp(14), bp(15)], t3=True)

    xf = x.reshape(256, 2048)
    w1 = jnp.pad(head_proj_w, ((0, 0), (0, 67)))
    s1 = jnp.pad(head_proj_scale, (0, 67)).reshape(1, 128)
    t1 = jnp.pad(head_proj_shift, (0, 67)).reshape(1, 128)
    w2 = jnp.pad(final_w, ((0, 67), (0, 124)))
    s2 = jnp.pad(final_scale, (0, 124)).reshape(1, 128)
    t2 = jnp.pad(final_shift, (0, 124)).reshape(1, 128)
    out = _head(xf, w1, s1, t1, w2, s2, t2)
    return out[0:1, 0:4]
